# baseline (speedup 1.0000x reference)
import functools

import jax
import jax.numpy as jnp
from jax import lax
from jax.experimental import pallas as pl
from jax.experimental.pallas import tpu as pltpu
from jax.experimental.pallas import tpu_sc as plsc

D = 64
NNEG = 4
NITEM = NNEG + 1


def _sc_worker_count():
    try:
        info = plsc.get_sparse_core_info()
        return info.num_cores, info.num_subcores
    except Exception:
        return 2, 16


@functools.lru_cache(maxsize=None)
def _make_gather(B, nc, ns):
    nw = nc * ns
    bpw = B // nw
    ipw = NITEM * bpw
    nchunk = NITEM
    mesh = plsc.VectorSubcoreMesh(core_axis_name="c", subcore_axis_name="s")

    @functools.partial(
        pl.kernel,
        mesh=mesh,
        compiler_params=pltpu.CompilerParams(use_tc_tiling_on_sc=False),
        out_type=[
            jax.ShapeDtypeStruct((B, D), jnp.float32),
            jax.ShapeDtypeStruct((B, D), jnp.float32),
            jax.ShapeDtypeStruct((NITEM * B, D), jnp.float32),
            jax.ShapeDtypeStruct((NITEM * B, D), jnp.float32),
        ],
        scratch_types=[
            pltpu.VMEM((bpw,), jnp.int32),
            pltpu.VMEM((ipw,), jnp.int32),
            pltpu.VMEM((bpw, D), jnp.float32),
            pltpu.VMEM((bpw, D), jnp.float32),
            pltpu.VMEM((ipw, D), jnp.float32),
            pltpu.VMEM((ipw, D), jnp.float32),
            pltpu.SemaphoreType.DMA,
        ],
    )
    def gk(user1d, items1d, mfu_t, mfi_t, mlu_t, mli_t,
           out_mfu, out_mlu, out_mfi, out_mli,
           idx_u, idx_it, r_mfu, r_mlu, r_mfi, r_mli, sem):
        wid = lax.axis_index("s") * nc + lax.axis_index("c")
        pltpu.sync_copy(user1d.at[pl.ds(wid * bpw, bpw)], idx_u)
        pltpu.sync_copy(items1d.at[pl.ds(wid * ipw, ipw)], idx_it)
        cps = [
            pltpu.async_copy(mfu_t.at[idx_u], r_mfu, sem),
            pltpu.async_copy(mlu_t.at[idx_u], r_mlu, sem),
        ]
        for j in range(nchunk):
            src = idx_it.at[pl.ds(j * bpw, bpw)]
            dst = pl.ds(j * bpw, bpw)
            cps.append(pltpu.async_copy(mfi_t.at[src], r_mfi.at[dst], sem))
            cps.append(pltpu.async_copy(mli_t.at[src], r_mli.at[dst], sem))
        for c in cps:
            c.wait()
        pltpu.sync_copy(r_mfu, out_mfu.at[pl.ds(wid * bpw, bpw)])
        pltpu.sync_copy(r_mlu, out_mlu.at[pl.ds(wid * bpw, bpw)])
        pltpu.sync_copy(r_mfi, out_mfi.at[pl.ds(wid * ipw, ipw)])
        pltpu.sync_copy(r_mli, out_mli.at[pl.ds(wid * ipw, ipw)])

    return gk


def _dense_body(mfu_ref, mlu_ref, mfi_ref, mli_ref,
                w1_ref, b1_ref, w2_ref, b2_ref, w3_ref, b3_ref,
                w4_ref, b4_ref, wd_ref, bd_ref, out_ref):
    r = mfu_ref.shape[0]
    mfu = mfu_ref[...]
    mlu = mlu_ref[...]
    sig_parts = []
    x_parts = []
    for k in range(NITEM):
        sig_parts.append(jax.nn.sigmoid(mfu * mfi_ref[k]))
        x_parts.append(jnp.concatenate([mlu, mli_ref[k]], axis=1))
    sig = jnp.concatenate(sig_parts, axis=0)
    x = jnp.concatenate(x_parts, axis=0)
    for w_ref, b_ref in ((w1_ref, b1_ref), (w2_ref, b2_ref),
                         (w3_ref, b3_ref), (w4_ref, b4_ref)):
        x = jnp.maximum(
            jnp.dot(x, w_ref[...], preferred_element_type=jnp.float32)
            + b_ref[...], 0.0)
    feat = jnp.concatenate([sig, x], axis=1)
    scores = jnp.dot(feat, wd_ref[...], preferred_element_type=jnp.float32) \
        + bd_ref[...]
    s = [scores[k * r:(k + 1) * r] for k in range(NITEM)]
    out_ref[...] = jnp.concatenate(
        [s[0], s[0], s[0], s[0], s[1], s[2], s[3], s[4]], axis=1)


def _dense(mfu, mlu, mfi3, mli3, W1, b1, W2, b2, W3, b3, W4, b4, Wd, bd):
    B = mfu.shape[0]
    R = 512
    grid = (B // R,)
    full = lambda shape: pl.BlockSpec(shape, lambda i: tuple(0 for _ in shape))
    in_specs = [
        pl.BlockSpec((R, D), lambda i: (i, 0)),
        pl.BlockSpec((R, D), lambda i: (i, 0)),
        pl.BlockSpec((NITEM, R, D), lambda i: (0, i, 0)),
        pl.BlockSpec((NITEM, R, D), lambda i: (0, i, 0)),
        full(W1.shape), full((1, b1.shape[0])),
        full(W2.shape), full((1, b2.shape[0])),
        full(W3.shape), full((1, b3.shape[0])),
        full(W4.shape), full((1, b4.shape[0])),
        full(Wd.shape), full((1, 1)),
    ]
    return pl.pallas_call(
        _dense_body,
        grid=grid,
        in_specs=in_specs,
        out_specs=pl.BlockSpec((R, 2 * NNEG), lambda i: (i, 0)),
        out_shape=jax.ShapeDtypeStruct((B, 2 * NNEG), jnp.float32),
    )(mfu, mlu, mfi3, mli3,
      W1, b1.reshape(1, -1), W2, b2.reshape(1, -1),
      W3, b3.reshape(1, -1), W4, b4.reshape(1, -1),
      Wd, bd.reshape(1, 1))


def kernel(user, pos_item, neg_item, mf_user_table, mf_item_table,
           mlp_user_table, mlp_item_table,
           W1, b1, W2, b2, W3, b3, W4, b4, Wd, bd):
    B = user.shape[0]
    nc, ns = _sc_worker_count()
    nw = nc * ns
    bpw = B // nw
    user1d = user.astype(jnp.int32)
    items = jnp.concatenate(
        [pos_item.astype(jnp.int32)[None, :], neg_item.astype(jnp.int32).T],
        axis=0)
    items1d = items.reshape(NITEM * B)
    gk = _make_gather(B, nc, ns)
    mfu, mlu, mfi, mli = gk(user1d, items1d, mf_user_table, mf_item_table,
                            mlp_user_table, mlp_item_table)
    mfi3 = mfi.reshape(NITEM, B, D)
    mli3 = mli.reshape(NITEM, B, D)
    return _dense(mfu, mlu, mfi3, mli3,
                  W1, b1, W2, b2, W3, b3, W4, b4, Wd, bd)
